# baseline (speedup 1.0000x reference)
import functools

import jax
import jax.numpy as jnp
from jax import lax
from jax.experimental import pallas as pl
from jax.experimental.pallas import tpu as pltpu
from jax.experimental.pallas import tpu_sc as plsc

B = 128
N = 32768
A = 4
NC = 2
NS = 16
NW = NC * NS
BPW = B // NW
L = 16
CH = 512
NCH = N // CH
VPC = CH // L
BIG = 1 << 20


def _sreduce(vec, init, op):
    acc = init
    for i in range(L):
        acc = op(acc, vec[i])
    return acc


def _planner_body(ev_hbm, act_hbm, out_hbm, row_v, cmax_v, gbuf_v, obuf_v):
    wid = lax.axis_index("s") * NC + lax.axis_index("c")
    iota = lax.iota(jnp.int32, L)

    for bb in range(BPW):
        b = wid * BPW + bb
        pltpu.sync_copy(ev_hbm.at[pl.ds(pl.multiple_of(b * N, L), N)], row_v)

        def chunk_body(c, _):
            base = c * CH
            m0 = row_v[pl.ds(base, L)]
            m1 = row_v[pl.ds(base + L, L)]
            m2 = row_v[pl.ds(base + 2 * L, L)]
            m3 = row_v[pl.ds(base + 3 * L, L)]
            for j in range(4, VPC, 4):
                m0 = jnp.maximum(m0, row_v[pl.ds(base + j * L, L)])
                m1 = jnp.maximum(m1, row_v[pl.ds(base + (j + 1) * L, L)])
                m2 = jnp.maximum(m2, row_v[pl.ds(base + (j + 2) * L, L)])
                m3 = jnp.maximum(m3, row_v[pl.ds(base + (j + 3) * L, L)])
            mm = jnp.maximum(jnp.maximum(m0, m1), jnp.maximum(m2, m3))
            cmax_v[pl.ds(c * L, L)] = mm
            return 0

        lax.fori_loop(0, NCH, chunk_body, 0)

        def bmax_body(k, acc):
            return jnp.maximum(acc, cmax_v[pl.ds(k * L, L)])

        macc = lax.fori_loop(
            0, NCH, bmax_body, jnp.full((L,), -jnp.inf, jnp.float32)
        )
        m = _sreduce(macc, jnp.float32(-jnp.inf), jnp.maximum)

        def bfind_body(k, acc):
            v = cmax_v[pl.ds(k * L, L)]
            return jnp.where(v == m, jnp.minimum(acc, k), acc)

        cacc = lax.fori_loop(
            0, NCH, bfind_body, jnp.full((L,), BIG, jnp.int32)
        )
        cstar = _sreduce(cacc, jnp.int32(BIG), jnp.minimum)

        cbase = cstar * CH

        def cfind_body(j, acc):
            v = row_v[pl.ds(cbase + j * L, L)]
            return jnp.where(v == m, jnp.minimum(acc, j), acc)

        jacc = lax.fori_loop(
            0, VPC, cfind_body, jnp.full((L,), BIG, jnp.int32)
        )
        rel = _sreduce(jacc * L + iota, jnp.int32(BIG * L * 2), jnp.minimum)
        idx = cbase + rel

        flat = idx * A
        off = lax.bitwise_and(flat, 7)
        start = flat - off
        src = pl.multiple_of(b * (N * A) + start, 8)
        pltpu.sync_copy(act_hbm.at[pl.ds(src, 8)], gbuf_v.at[pl.ds(0, 8)])
        v = gbuf_v[...]
        lo = jnp.where(iota == 0, v[0],
             jnp.where(iota == 1, v[1],
             jnp.where(iota == 2, v[2], v[3])))
        hi = jnp.where(iota == 0, v[4],
             jnp.where(iota == 1, v[5],
             jnp.where(iota == 2, v[6], v[7])))
        obuf_v[...] = jnp.where(off == 0, lo, hi)
        pltpu.sync_copy(obuf_v, out_hbm.at[pl.ds(pl.multiple_of(b * L, L), L)])


_planner = functools.partial(
    pl.kernel,
    out_type=jax.ShapeDtypeStruct((B * L,), jnp.float32),
    mesh=plsc.VectorSubcoreMesh(core_axis_name="c", subcore_axis_name="s"),
    scratch_types=[
        pltpu.VMEM((N,), jnp.float32),
        pltpu.VMEM((NCH * L,), jnp.float32),
        pltpu.VMEM((L,), jnp.float32),
        pltpu.VMEM((L,), jnp.float32),
    ],
)(_planner_body)


def kernel(estimated_value, action):
    ev = estimated_value.reshape(B * N)
    act = action.reshape(B * N * A)
    out = _planner(ev, act)
    return out.reshape(B, L)[:, :A]


# speedup vs baseline: 132.1631x; 132.1631x over previous
import functools

import jax
import jax.numpy as jnp
from jax import lax
from jax.experimental import pallas as pl
from jax.experimental.pallas import tpu as pltpu
from jax.experimental.pallas import tpu_sc as plsc

B = 128
N = 32768
A = 4
NC = 2
NS = 16
NW = NC * NS
BPW = B // NW
L = 16
CH = 512
NCH = N // CH
VPC = CH // L
BIG = 1 << 20


def _sreduce(vec, init, op):
    acc = init
    for i in range(L):
        acc = op(acc, vec[i])
    return acc


def _planner_body(ev_hbm, act_hbm, out_hbm, row_v, cmax_v, gbuf_v, obuf_v):
    wid = lax.axis_index("s") * NC + lax.axis_index("c")
    iota = lax.iota(jnp.int32, L)

    for bb in range(BPW):
        b = wid * BPW + bb
        pltpu.sync_copy(ev_hbm.at[pl.ds(pl.multiple_of(b * N, L), N)], row_v)

        def chunk_body(c, _):
            base = c * CH
            m0 = row_v[pl.ds(base, L)]
            m1 = row_v[pl.ds(base + L, L)]
            m2 = row_v[pl.ds(base + 2 * L, L)]
            m3 = row_v[pl.ds(base + 3 * L, L)]
            for j in range(4, VPC, 4):
                m0 = jnp.maximum(m0, row_v[pl.ds(base + j * L, L)])
                m1 = jnp.maximum(m1, row_v[pl.ds(base + (j + 1) * L, L)])
                m2 = jnp.maximum(m2, row_v[pl.ds(base + (j + 2) * L, L)])
                m3 = jnp.maximum(m3, row_v[pl.ds(base + (j + 3) * L, L)])
            mm = jnp.maximum(jnp.maximum(m0, m1), jnp.maximum(m2, m3))
            cmax_v[pl.ds(c * L, L)] = mm
            return 0

        lax.fori_loop(0, NCH, chunk_body, 0)

        def bmax_body(k, acc):
            return jnp.maximum(acc, cmax_v[pl.ds(k * L, L)])

        macc = lax.fori_loop(
            0, NCH, bmax_body, jnp.full((L,), -jnp.inf, jnp.float32)
        )
        m = _sreduce(macc, jnp.float32(-jnp.inf), jnp.maximum)

        def bfind_body(k, acc):
            v = cmax_v[pl.ds(k * L, L)]
            return jnp.where(v == m, jnp.minimum(acc, k), acc)

        cacc = lax.fori_loop(
            0, NCH, bfind_body, jnp.full((L,), BIG, jnp.int32)
        )
        cstar = _sreduce(cacc, jnp.int32(BIG), jnp.minimum)

        cbase = cstar * CH

        def cfind_body(j, acc):
            v = row_v[pl.ds(cbase + j * L, L)]
            return jnp.where(v == m, jnp.minimum(acc, j), acc)

        jacc = lax.fori_loop(
            0, VPC, cfind_body, jnp.full((L,), BIG, jnp.int32)
        )
        rel = _sreduce(jacc * L + iota, jnp.int32(BIG * L * 2), jnp.minimum)
        idx = cbase + rel

        grp = b * (N * A) + lax.shift_right_logical(idx, 7) * 512
        pltpu.sync_copy(act_hbm.at[pl.ds(pl.multiple_of(grp, 16), 512)],
                        gbuf_v)
        off16 = lax.bitwise_and(idx, 127) - lax.bitwise_and(idx, 15)
        lane = lax.bitwise_and(idx, 15)
        eq = iota == lane
        s = []
        for a in range(A):
            va = gbuf_v[pl.ds(pl.multiple_of(a * 128 + off16, 16), L)]
            s.append(_sreduce(jnp.where(eq, va, jnp.float32(0.0)),
                              jnp.float32(0.0), jnp.add))
        obuf_v[...] = jnp.where(iota == 0, s[0],
                      jnp.where(iota == 1, s[1],
                      jnp.where(iota == 2, s[2], s[3])))
        pltpu.sync_copy(obuf_v, out_hbm.at[pl.ds(pl.multiple_of(b * L, L), L)])


_planner = functools.partial(
    pl.kernel,
    out_type=jax.ShapeDtypeStruct((B * L,), jnp.float32),
    mesh=plsc.VectorSubcoreMesh(core_axis_name="c", subcore_axis_name="s"),
    scratch_types=[
        pltpu.VMEM((N,), jnp.float32),
        pltpu.VMEM((NCH * L,), jnp.float32),
        pltpu.VMEM((512,), jnp.float32),
        pltpu.VMEM((L,), jnp.float32),
    ],
)(_planner_body)


def kernel(estimated_value, action):
    ev = estimated_value.reshape(B * N)
    act = action.reshape(B, N // 128, 128, A)
    act = act.transpose(0, 1, 3, 2).reshape(B * N * A)
    out = _planner(ev, act)
    return out.reshape(B, L)[:, :A]
